# baseline (speedup 1.0000x reference)
import jax
import jax.numpy as jnp
from jax import lax
from jax.experimental import pallas as pl
from jax.experimental.pallas import tpu as pltpu
from jax.experimental.pallas import tpu_sc as plsc

NMERS = 6
HID = 768
LAB = 16
B = 16
S = 512
P = S + NMERS - 1

NC = 2
NS = 16
WPR = (NC * NS) // B
CHUNK = 264
PPAD = WPR * CHUNK


TCROWS = 8


def _proj_kernel(emb_ref, wt_ref, out_ref):
    for r in range(TCROWS):
        out_ref[r] = jnp.dot(emb_ref[r], wt_ref[:],
                             preferred_element_type=jnp.float32)


def _sc_unfold(proj_hbm, mask_hbm, b_hbm, out_hbm,
               prow_v, mrow_v, obuf_v, bvec_v):
    c = lax.axis_index("c")
    s = lax.axis_index("s")
    wid = s * NC + c
    row = wid // WPR
    part = wid % WPR
    qlo = part * CHUNK
    qhi = qlo + CHUNK

    pltpu.sync_copy(proj_hbm.at[row], prow_v)
    pltpu.sync_copy(mask_hbm.at[row], mrow_v)
    pltpu.sync_copy(b_hbm, bvec_v)
    bvec = bvec_v[...]

    def _msum(k, acc):
        return acc + mrow_v[pl.ds(k * LAB, LAB)]

    macc = lax.fori_loop(0, S // LAB, _msum, jnp.zeros((LAB,), jnp.int32),
                         unroll=8)
    L = macc[0]
    for k in range(1, LAB):
        L = L + macc[k]
    nc = jnp.maximum(L - 2, 1)

    def clampq(x):
        return jnp.clip(x, qlo, qhi)

    def pm(i):
        ii = jnp.clip(i, 0, S - 1)
        f = jnp.where((i >= 1) & (i <= nc), 1.0, 0.0).astype(jnp.float32)
        return prow_v[ii] * f

    def _general(q, acc):
        acc = acc + pm(q)
        lo = jnp.maximum(1, q - 5)
        hi = jnp.minimum(nc, q)
        den = jnp.maximum(hi - lo + 1, 1)
        rcp = jnp.float32(1.0)
        for d in range(2, NMERS + 1):
            rcp = jnp.where(den == d, jnp.float32(1.0 / d), rcp)
        obuf_v[q - qlo] = acc * rcp + bvec
        return acc - pm(q - 5)

    def _fast(q, acc):
        acc = acc + prow_v[q]
        obuf_v[q - qlo] = acc * jnp.float32(1.0 / NMERS) + bvec
        return acc - prow_v[q - 5]

    def _fill(q, acc):
        obuf_v[q - qlo] = bvec
        return acc

    head_end = clampq(NMERS)
    mid_end = clampq(nc + 1)
    edge_lo = clampq(jnp.maximum(NMERS, nc + 1))
    edge_hi = clampq(nc + NMERS)
    fill_lo = clampq(nc + NMERS)

    acc0 = jnp.zeros((LAB,), jnp.float32)
    for k in range(5):
        acc0 = acc0 + pm(qlo - 5 + k)

    acc0 = lax.fori_loop(qlo, head_end, _general, acc0)
    acc0 = lax.fori_loop(head_end, mid_end, _fast, acc0)
    acc0 = lax.fori_loop(edge_lo, edge_hi, _general, acc0)
    lax.fori_loop(fill_lo, qhi, _fill, acc0)

    @pl.when(part == 0)
    def _():
        obuf_v[0] = prow_v[0] + bvec

    eidx = jnp.where(L >= 1, L - 1, S - 1)
    qe = L + 4

    @pl.when((qe >= qlo) & (qe < qhi))
    def _():
        obuf_v[qe - qlo] = prow_v[eidx] + bvec

    pltpu.sync_copy(obuf_v, out_hbm.at[row, pl.ds(qlo, CHUNK)])


@jax.jit
def kernel(outputs, attention_mask, W_dec, b_dec):
    emb = outputs[0]
    wt = W_dec.T

    proj = pl.pallas_call(
        _proj_kernel,
        grid=(B // TCROWS,),
        in_specs=[
            pl.BlockSpec((TCROWS, S, HID), lambda b: (b, 0, 0)),
            pl.BlockSpec((HID, LAB), lambda b: (0, 0)),
        ],
        out_specs=pl.BlockSpec((TCROWS, S, LAB), lambda b: (b, 0, 0)),
        out_shape=jax.ShapeDtypeStruct((B, S, LAB), jnp.float32),
    )(emb, wt)

    sc_unfold = pl.kernel(
        _sc_unfold,
        out_type=jax.ShapeDtypeStruct((B, PPAD, LAB), jnp.float32),
        mesh=plsc.VectorSubcoreMesh(
            core_axis_name="c", subcore_axis_name="s",
            num_cores=NC, num_subcores=NS),
        scratch_types=[
            pltpu.VMEM((S, LAB), jnp.float32),
            pltpu.VMEM((S,), jnp.int32),
            pltpu.VMEM((CHUNK, LAB), jnp.float32),
            pltpu.VMEM((LAB,), jnp.float32),
        ],
    )

    out = sc_unfold(proj, attention_mask, b_dec)
    return out[:, :P, :]


# speedup vs baseline: 1.0348x; 1.0348x over previous
import jax
import jax.numpy as jnp
from jax import lax
from jax.experimental import pallas as pl
from jax.experimental.pallas import tpu as pltpu
from jax.experimental.pallas import tpu_sc as plsc

NMERS = 6
HID = 768
LAB = 16
B = 16
S = 512
P = S + NMERS - 1

NC = 2
NS = 16
WPR = (NC * NS) // B
CHUNK = 264
PPAD = WPR * CHUNK


TCROWS = 4


def _proj_kernel(emb_ref, wt_ref, out_ref):
    for r in range(TCROWS):
        out_ref[r] = jnp.dot(emb_ref[r], wt_ref[:],
                             preferred_element_type=jnp.float32)


def _sc_unfold(proj_hbm, mask_hbm, b_hbm, out_hbm,
               prow_v, mrow_v, obuf_v, bvec_v, sem_p, sem_m, sem_b):
    c = lax.axis_index("c")
    s = lax.axis_index("s")
    wid = s * NC + c
    row = wid // WPR
    part = wid % WPR
    qlo = part * CHUNK
    qhi = qlo + CHUNK

    cp_p = pltpu.async_copy(proj_hbm.at[row], prow_v, sem_p)
    cp_m = pltpu.async_copy(mask_hbm.at[row], mrow_v, sem_m)
    cp_b = pltpu.async_copy(b_hbm, bvec_v, sem_b)
    cp_m.wait()

    def _msum(k, acc):
        return acc + mrow_v[pl.ds(k * LAB, LAB)]

    macc = lax.fori_loop(0, S // LAB, _msum, jnp.zeros((LAB,), jnp.int32),
                         unroll=8)
    L = macc[0]
    for k in range(1, LAB):
        L = L + macc[k]
    nc = jnp.maximum(L - 2, 1)

    def clampq(x):
        return jnp.clip(x, qlo, qhi)

    def pm(i):
        ii = jnp.clip(i, 0, S - 1)
        f = jnp.where((i >= 1) & (i <= nc), 1.0, 0.0).astype(jnp.float32)
        return prow_v[ii] * f

    def _general(q, acc):
        acc = acc + pm(q)
        lo = jnp.maximum(1, q - 5)
        hi = jnp.minimum(nc, q)
        den = jnp.maximum(hi - lo + 1, 1)
        rcp = jnp.float32(1.0)
        for d in range(2, NMERS + 1):
            rcp = jnp.where(den == d, jnp.float32(1.0 / d), rcp)
        obuf_v[q - qlo] = acc * rcp + bvec
        return acc - pm(q - 5)

    def _fast(q, acc):
        acc = acc + prow_v[q]
        obuf_v[q - qlo] = acc * jnp.float32(1.0 / NMERS) + bvec
        return acc - prow_v[q - 5]

    def _fill(q, acc):
        obuf_v[q - qlo] = bvec
        return acc

    head_end = clampq(NMERS)
    mid_end = clampq(nc + 1)
    edge_lo = clampq(jnp.maximum(NMERS, nc + 1))
    edge_hi = clampq(nc + NMERS)
    fill_lo = clampq(nc + NMERS)

    cp_p.wait()
    cp_b.wait()
    bvec = bvec_v[...]

    acc0 = jnp.zeros((LAB,), jnp.float32)
    for k in range(5):
        acc0 = acc0 + pm(qlo - 5 + k)

    acc0 = lax.fori_loop(qlo, head_end, _general, acc0)
    acc0 = lax.fori_loop(head_end, mid_end, _fast, acc0)
    acc0 = lax.fori_loop(edge_lo, edge_hi, _general, acc0)
    lax.fori_loop(fill_lo, qhi, _fill, acc0)

    @pl.when(part == 0)
    def _():
        obuf_v[0] = prow_v[0] + bvec

    eidx = jnp.where(L >= 1, L - 1, S - 1)
    qe = L + 4

    @pl.when((qe >= qlo) & (qe < qhi))
    def _():
        obuf_v[qe - qlo] = prow_v[eidx] + bvec

    pltpu.sync_copy(obuf_v, out_hbm.at[row, pl.ds(qlo, CHUNK)])


@jax.jit
def kernel(outputs, attention_mask, W_dec, b_dec):
    emb = outputs[0]
    wt = W_dec.T

    proj = pl.pallas_call(
        _proj_kernel,
        grid=(B // TCROWS,),
        in_specs=[
            pl.BlockSpec((TCROWS, S, HID), lambda b: (b, 0, 0)),
            pl.BlockSpec((HID, LAB), lambda b: (0, 0)),
        ],
        out_specs=pl.BlockSpec((TCROWS, S, LAB), lambda b: (b, 0, 0)),
        out_shape=jax.ShapeDtypeStruct((B, S, LAB), jnp.float32),
    )(emb, wt)

    sc_unfold = pl.kernel(
        _sc_unfold,
        out_type=jax.ShapeDtypeStruct((B, PPAD, LAB), jnp.float32),
        mesh=plsc.VectorSubcoreMesh(
            core_axis_name="c", subcore_axis_name="s",
            num_cores=NC, num_subcores=NS),
        scratch_types=[
            pltpu.VMEM((S, LAB), jnp.float32),
            pltpu.VMEM((S,), jnp.int32),
            pltpu.VMEM((CHUNK, LAB), jnp.float32),
            pltpu.VMEM((LAB,), jnp.float32),
            pltpu.SemaphoreType.DMA,
            pltpu.SemaphoreType.DMA,
            pltpu.SemaphoreType.DMA,
        ],
    )

    out = sc_unfold(proj, attention_mask, b_dec)
    return out[:, :P, :]


# speedup vs baseline: 1.0634x; 1.0276x over previous
import jax
import jax.numpy as jnp
from jax import lax
from jax.experimental import pallas as pl
from jax.experimental.pallas import tpu as pltpu
from jax.experimental.pallas import tpu_sc as plsc

NMERS = 6
HID = 768
LAB = 16
B = 16
S = 512
P = S + NMERS - 1

NC = 2
NS = 16
WPR = (NC * NS) // B
CHUNK = 264
PPAD = WPR * CHUNK


TCROWS = 4


def _proj_kernel(emb_ref, wt_ref, out_ref):
    e = emb_ref[...].reshape(TCROWS * S, HID)
    out_ref[...] = jnp.dot(e, wt_ref[:], preferred_element_type=jnp.float32
                           ).reshape(TCROWS, S, LAB)


def _sc_unfold(proj_hbm, mask_hbm, b_hbm, out_hbm,
               prow_v, mrow_v, obuf_v, bvec_v, sem_p, sem_m, sem_b):
    c = lax.axis_index("c")
    s = lax.axis_index("s")
    wid = s * NC + c
    row = wid // WPR
    part = wid % WPR
    qlo = part * CHUNK
    qhi = qlo + CHUNK

    cp_p = pltpu.async_copy(proj_hbm.at[row], prow_v, sem_p)
    cp_m = pltpu.async_copy(mask_hbm.at[row], mrow_v, sem_m)
    cp_b = pltpu.async_copy(b_hbm, bvec_v, sem_b)
    cp_m.wait()

    def _msum(k, acc):
        return acc + mrow_v[pl.ds(k * LAB, LAB)]

    macc = lax.fori_loop(0, S // LAB, _msum, jnp.zeros((LAB,), jnp.int32),
                         unroll=8)
    L = macc[0]
    for k in range(1, LAB):
        L = L + macc[k]
    nc = jnp.maximum(L - 2, 1)

    def clampq(x):
        return jnp.clip(x, qlo, qhi)

    def pm(i):
        ii = jnp.clip(i, 0, S - 1)
        f = jnp.where((i >= 1) & (i <= nc), 1.0, 0.0).astype(jnp.float32)
        return prow_v[ii] * f

    def _general(q, acc):
        acc = acc + pm(q)
        lo = jnp.maximum(1, q - 5)
        hi = jnp.minimum(nc, q)
        den = jnp.maximum(hi - lo + 1, 1)
        rcp = jnp.float32(1.0)
        for d in range(2, NMERS + 1):
            rcp = jnp.where(den == d, jnp.float32(1.0 / d), rcp)
        obuf_v[q - qlo] = acc * rcp + bvec
        return acc - pm(q - 5)

    def _fast(q, acc):
        acc = acc + prow_v[q]
        obuf_v[q - qlo] = acc * jnp.float32(1.0 / NMERS) + bvec
        return acc - prow_v[q - 5]

    def _fill(q, acc):
        obuf_v[q - qlo] = bvec
        return acc

    head_end = clampq(NMERS)
    mid_end = clampq(nc + 1)
    edge_lo = clampq(jnp.maximum(NMERS, nc + 1))
    edge_hi = clampq(nc + NMERS)
    fill_lo = clampq(nc + NMERS)

    cp_p.wait()
    cp_b.wait()
    bvec = bvec_v[...]

    acc0 = jnp.zeros((LAB,), jnp.float32)
    for k in range(5):
        acc0 = acc0 + pm(qlo - 5 + k)

    acc0 = lax.fori_loop(qlo, head_end, _general, acc0)
    acc0 = lax.fori_loop(head_end, mid_end, _fast, acc0)
    acc0 = lax.fori_loop(edge_lo, edge_hi, _general, acc0)
    lax.fori_loop(fill_lo, qhi, _fill, acc0)

    @pl.when(part == 0)
    def _():
        obuf_v[0] = prow_v[0] + bvec

    eidx = jnp.where(L >= 1, L - 1, S - 1)
    qe = L + 4

    @pl.when((qe >= qlo) & (qe < qhi))
    def _():
        obuf_v[qe - qlo] = prow_v[eidx] + bvec

    @pl.when(part == 0)
    def _():
        pltpu.sync_copy(obuf_v, out_hbm.at[row, pl.ds(0, CHUNK)])

    @pl.when(part == 1)
    def _():
        pltpu.sync_copy(obuf_v.at[pl.ds(0, P - CHUNK)],
                        out_hbm.at[row, pl.ds(CHUNK, P - CHUNK)])


@jax.jit
def kernel(outputs, attention_mask, W_dec, b_dec):
    emb = outputs[0]
    wt = W_dec.T

    proj = pl.pallas_call(
        _proj_kernel,
        grid=(B // TCROWS,),
        in_specs=[
            pl.BlockSpec((TCROWS, S, HID), lambda b: (b, 0, 0)),
            pl.BlockSpec((HID, LAB), lambda b: (0, 0)),
        ],
        out_specs=pl.BlockSpec((TCROWS, S, LAB), lambda b: (b, 0, 0)),
        out_shape=jax.ShapeDtypeStruct((B, S, LAB), jnp.float32),
    )(emb, wt)

    sc_unfold = pl.kernel(
        _sc_unfold,
        out_type=jax.ShapeDtypeStruct((B, P, LAB), jnp.float32),
        mesh=plsc.VectorSubcoreMesh(
            core_axis_name="c", subcore_axis_name="s",
            num_cores=NC, num_subcores=NS),
        scratch_types=[
            pltpu.VMEM((S, LAB), jnp.float32),
            pltpu.VMEM((S,), jnp.int32),
            pltpu.VMEM((CHUNK, LAB), jnp.float32),
            pltpu.VMEM((LAB,), jnp.float32),
            pltpu.SemaphoreType.DMA,
            pltpu.SemaphoreType.DMA,
            pltpu.SemaphoreType.DMA,
        ],
    )

    return sc_unfold(proj, attention_mask, b_dec)
